# Initial kernel scaffold; baseline (speedup 1.0000x reference)
import functools

import jax
import jax.numpy as jnp
from jax import lax
from jax.experimental import pallas as pl
from jax.experimental.pallas import tpu as pltpu
from jax.experimental.pallas import tpu_sc as plsc

EPS = 1e-5


_LN_BLK = 1024


def _ln_tables_body(dx_ref, proc_ref, visit_ref, g_ref, b_ref,
                    dx_out, proc_out, visit_out):
    g = g_ref[...]
    b = b_ref[...]
    for src, dst in ((dx_ref, dx_out), (proc_ref, proc_out), (visit_ref, visit_out)):
        x = src[...]
        m = jnp.mean(x, axis=-1, keepdims=True)
        v = jnp.mean((x - m) ** 2, axis=-1, keepdims=True)
        dst[...] = (x - m) * lax.rsqrt(v + EPS) * g + b


def _ln_tables(dx_table, proc_table, visit_table, ln_gamma, ln_beta):
    v1, d = dx_table.shape
    n_blk = pl.cdiv(v1, _LN_BLK)
    row_spec = pl.BlockSpec((_LN_BLK, d), lambda i: (i, 0))
    one_spec = pl.BlockSpec((1, d), lambda i: (0, 0))
    return pl.pallas_call(
        _ln_tables_body,
        grid=(n_blk,),
        in_specs=[row_spec, row_spec, one_spec, one_spec, one_spec],
        out_specs=[row_spec, row_spec, one_spec],
        out_shape=[
            jax.ShapeDtypeStruct((v1, d), jnp.float32),
            jax.ShapeDtypeStruct((v1, d), jnp.float32),
            jax.ShapeDtypeStruct((1, d), jnp.float32),
        ],
    )(dx_table, proc_table, visit_table,
      ln_gamma.reshape(1, d), ln_beta.reshape(1, d))


_SUB = 128
_CHUNK = 512
_NSUB = _CHUNK // _SUB


def _gather_body(n_chunks, per_w, d,
                 dx_tab, dx_idx, proc_tab, proc_idx, dx_out, proc_out,
                 idx_buf, row_buf, sg0, sg1, so0, so1):
    nc = 2
    wid = lax.axis_index("s") * nc + lax.axis_index("c")
    base = wid * per_w
    sems_g = (sg0, sg1)
    sems_o = (so0, so1)

    def run_table(tab, idxh, outh):
        def row0(c):
            return pl.multiple_of(base + c * _CHUNK, _CHUNK)

        def fire(slot, c):
            pltpu.sync_copy(idxh.at[pl.ds(row0(c), _CHUNK)], idx_buf.at[slot])
            for j in range(_NSUB):
                pltpu.async_copy(
                    tab.at[idx_buf.at[slot, pl.ds(j * _SUB, _SUB)]],
                    row_buf.at[slot, pl.ds(j * _SUB, _SUB), :],
                    sems_g[slot])

        def drain_store(slot, c):
            for j in range(_NSUB):
                pltpu.make_async_copy(
                    tab.at[idx_buf.at[slot, pl.ds(j * _SUB, _SUB)]],
                    row_buf.at[slot, pl.ds(j * _SUB, _SUB), :],
                    sems_g[slot]).wait()
            pltpu.async_copy(row_buf.at[slot],
                             outh.at[pl.ds(row0(c), _CHUNK), :],
                             sems_o[slot])

        def retire(slot, c):
            pltpu.make_async_copy(row_buf.at[slot],
                                  outh.at[pl.ds(row0(c), _CHUNK), :],
                                  sems_o[slot]).wait()

        fire(0, 0)
        fire(1, 1)
        drain_store(0, 0)

        @pl.loop(2, n_chunks, step=2)
        def _steady(c):
            retire(0, c - 2)
            fire(0, c)
            drain_store(1, c - 1)
            retire(1, c - 1)
            fire(1, c + 1)
            drain_store(0, c)

        drain_store(1, n_chunks - 1)
        retire(0, n_chunks - 2)
        retire(1, n_chunks - 1)

    run_table(dx_tab, dx_idx, dx_out)
    run_table(proc_tab, proc_idx, proc_out)


def _sc_gather(dx_ln, proc_ln, dx_idx, proc_idx):
    v1, d = dx_ln.shape
    (rows,) = dx_idx.shape
    nw = 32
    per_w = rows // nw
    n_chunks = per_w // _CHUNK
    mesh = plsc.VectorSubcoreMesh(core_axis_name="c", subcore_axis_name="s")
    run = pl.kernel(
        functools.partial(_gather_body, n_chunks, per_w, d),
        out_type=[
            jax.ShapeDtypeStruct((rows, d), jnp.float32),
            jax.ShapeDtypeStruct((rows, d), jnp.float32),
        ],
        mesh=mesh,
        scratch_types=[
            pltpu.VMEM((2, _CHUNK), jnp.int32),
            pltpu.VMEM((2, _CHUNK, d), jnp.float32),
            pltpu.SemaphoreType.DMA,
            pltpu.SemaphoreType.DMA,
            pltpu.SemaphoreType.DMA,
            pltpu.SemaphoreType.DMA,
        ],
    )
    return run(dx_ln, dx_idx, proc_ln, proc_idx)


def kernel(dx_table, proc_table, visit_table, ln_gamma, ln_beta,
           dx_ints, proc_ints):
    b, l = dx_ints.shape
    d = dx_table.shape[1]
    dx_ln, proc_ln, visit_ln = _ln_tables(
        dx_table, proc_table, visit_table, ln_gamma, ln_beta)
    dx_idx = dx_ints.reshape(-1).astype(jnp.int32)
    proc_idx = proc_ints.reshape(-1).astype(jnp.int32)
    dx_rows, proc_rows = _sc_gather(dx_ln, proc_ln, dx_idx, proc_idx)
    dx_emb = dx_rows.reshape(b, l, d)
    proc_emb = proc_rows.reshape(b, l, d)
    visit_emb = jnp.broadcast_to(visit_ln.reshape(1, 1, d), (b, 1, d))
    visit_mask = jnp.ones((b, 1), dtype=jnp.float32)
    return (dx_emb, proc_emb, visit_emb, visit_mask)


# speedup vs baseline: 4.0872x; 4.0872x over previous
import functools

import jax
import jax.numpy as jnp
from jax import lax
from jax.experimental import pallas as pl
from jax.experimental.pallas import tpu as pltpu
from jax.experimental.pallas import tpu_sc as plsc

EPS = 1e-5


_LN_BLK = 1024


def _ln_tables_body(dx_ref, proc_ref, visit_ref, g_ref, b_ref,
                    dx_out, proc_out, visit_out):
    g = g_ref[...]
    b = b_ref[...]
    for src, dst in ((dx_ref, dx_out), (proc_ref, proc_out), (visit_ref, visit_out)):
        x = src[...]
        m = jnp.mean(x, axis=-1, keepdims=True)
        v = jnp.mean((x - m) ** 2, axis=-1, keepdims=True)
        dst[...] = (x - m) * lax.rsqrt(v + EPS) * g + b


def _ln_tables(dx_table, proc_table, visit_table, ln_gamma, ln_beta):
    v1, d = dx_table.shape
    n_blk = pl.cdiv(v1, _LN_BLK)
    row_spec = pl.BlockSpec((_LN_BLK, d), lambda i: (i, 0))
    one_spec = pl.BlockSpec((1, d), lambda i: (0, 0))
    return pl.pallas_call(
        _ln_tables_body,
        grid=(n_blk,),
        in_specs=[row_spec, row_spec, one_spec, one_spec, one_spec],
        out_specs=[row_spec, row_spec, one_spec],
        out_shape=[
            jax.ShapeDtypeStruct((v1, d), jnp.float32),
            jax.ShapeDtypeStruct((v1, d), jnp.float32),
            jax.ShapeDtypeStruct((1, d), jnp.float32),
        ],
    )(dx_table, proc_table, visit_table,
      ln_gamma.reshape(1, d), ln_beta.reshape(1, d))


_SUB = 128
_CHUNK = 512
_NSUB = _CHUNK // _SUB


def _gather_body(n_chunks, per_w, d,
                 dx_tab, dx_idx, proc_tab, proc_idx, dx_out, proc_out,
                 idx_buf, row_buf, sg0, sg1, so0, so1):
    nc = 2
    wid = lax.axis_index("s") * nc + lax.axis_index("c")
    base = wid * per_w
    sems_g = (sg0, sg1)
    sems_o = (so0, so1)

    def run_table(tab, idxh, outh):
        def row0(c):
            return pl.multiple_of(base + c * _CHUNK, _CHUNK)

        def fire(slot, c):
            pltpu.sync_copy(idxh.at[pl.ds(row0(c), _CHUNK)], idx_buf.at[slot])
            for j in range(_NSUB):
                pltpu.async_copy(
                    tab.at[idx_buf.at[slot, pl.ds(j * _SUB, _SUB)]],
                    row_buf.at[slot, pl.ds(j * _SUB, _SUB), :],
                    sems_g[slot])

        def drain_store(slot, c):
            for j in range(_NSUB):
                pltpu.make_async_copy(
                    tab.at[idx_buf.at[slot, pl.ds(j * _SUB, _SUB)]],
                    row_buf.at[slot, pl.ds(j * _SUB, _SUB), :],
                    sems_g[slot]).wait()
            pltpu.async_copy(row_buf.at[slot],
                             outh.at[pl.ds(row0(c), _CHUNK), :],
                             sems_o[slot])

        def retire(slot, c):
            pltpu.make_async_copy(row_buf.at[slot],
                                  outh.at[pl.ds(row0(c), _CHUNK), :],
                                  sems_o[slot]).wait()

        fire(0, 0)
        fire(1, 1)
        drain_store(0, 0)

        @pl.loop(2, n_chunks, step=2)
        def _steady(c):
            retire(0, c - 2)
            fire(0, c)
            drain_store(1, c - 1)
            retire(1, c - 1)
            fire(1, c + 1)
            drain_store(0, c)

        drain_store(1, n_chunks - 1)
        retire(0, n_chunks - 2)
        retire(1, n_chunks - 1)

    run_table(dx_tab, dx_idx, dx_out)
    run_table(proc_tab, proc_idx, proc_out)


def _sc_gather(dx_ln, proc_ln, dx_idx, proc_idx):
    v1, d = dx_ln.shape
    (rows,) = dx_idx.shape
    nw = 32
    per_w = rows // nw
    n_chunks = per_w // _CHUNK
    mesh = plsc.VectorSubcoreMesh(core_axis_name="c", subcore_axis_name="s",
                                  num_cores=2, num_subcores=16)
    run = pl.kernel(
        functools.partial(_gather_body, n_chunks, per_w, d),
        out_type=[
            jax.ShapeDtypeStruct((rows, d), jnp.float32),
            jax.ShapeDtypeStruct((rows, d), jnp.float32),
        ],
        mesh=mesh,
        scratch_types=[
            pltpu.VMEM((2, _CHUNK), jnp.int32),
            pltpu.VMEM((2, _CHUNK, d), jnp.float32),
            pltpu.SemaphoreType.DMA,
            pltpu.SemaphoreType.DMA,
            pltpu.SemaphoreType.DMA,
            pltpu.SemaphoreType.DMA,
        ],
        compiler_params=pltpu.CompilerParams(use_tc_tiling_on_sc=False),
    )
    return run(dx_ln, dx_idx, proc_ln, proc_idx)


def kernel(dx_table, proc_table, visit_table, ln_gamma, ln_beta,
           dx_ints, proc_ints):
    b, l = dx_ints.shape
    d = dx_table.shape[1]
    dx_ln, proc_ln, visit_ln = _ln_tables(
        dx_table, proc_table, visit_table, ln_gamma, ln_beta)
    dx_idx = dx_ints.reshape(-1).astype(jnp.int32)
    proc_idx = proc_ints.reshape(-1).astype(jnp.int32)
    dx_rows, proc_rows = _sc_gather(dx_ln, proc_ln, dx_idx, proc_idx)
    dx_emb = dx_rows.reshape(b, l, d)
    proc_emb = proc_rows.reshape(b, l, d)
    visit_emb = jnp.broadcast_to(visit_ln.reshape(1, 1, d), (b, 1, d))
    visit_mask = jnp.ones((b, 1), dtype=jnp.float32)
    return (dx_emb, proc_emb, visit_emb, visit_mask)
